# baseline (speedup 1.0000x reference)
import functools

import jax
import jax.numpy as jnp
from jax import lax
from jax.experimental import pallas as pl
from jax.experimental.pallas import tpu as pltpu
from jax.experimental.pallas import tpu_sc as plsc

H = 128
N = 50000
NP = 50176
BLK = 1568
NBLK = 32
E_AST = 50000
E_VAR = 16000
E = 2 * E_AST + 2 * E_VAR
NTYPES = 4
LAYER_TIMESTEPS = [5, 2, 5, 2]

NCORES = 2
NSUB = 16
NW = NCORES * NSUB
RGN = 6272
NRGN = 8
DUMP = RGN
RGN_P = RGN + 8
CHK = 128
CHKB = 2 * CHK
EP = 134144
NCHB = EP // CHKB
PACK = 8192
TROWS = RGN // NSUB
SCH = 512
NCS = 258
EPS = NCS * SCH

@functools.lru_cache(maxsize=None)
def _mesh():
    return plsc.VectorSubcoreMesh(core_axis_name="c", subcore_axis_name="s",
                                  num_cores=NCORES, num_subcores=NSUB)


def _scatter_body(y_hbm, gidx_hbm, dloc_hbm, offs_hbm, zer_hbm, out_hbm,
                  gi_v, di_v, rows_v, offs_v, acc_sh, sem, ss):
    c = lax.axis_index("c")
    s = lax.axis_index("s")
    pltpu.async_copy(offs_hbm, offs_v, sem).wait()
    offv = offs_v[...]

    for r in range(NRGN):
        @pl.when(c == (r % NCORES))
        def _phase(r=r):
            base = s * TROWS
            pltpu.async_copy(zer_hbm, acc_sh.at[pl.ds(base, TROWS)],
                             sem).wait()
            plsc.subcore_barrier()

            @pl.loop(offv[r] + s * CHK, offv[r + 1], step=NSUB * CHK)
            def _chunk(off):
                off = pl.multiple_of(off, CHK)
                ca = pltpu.async_copy(gidx_hbm.at[pl.ds(off, CHK)], gi_v, sem)
                cb = pltpu.async_copy(dloc_hbm.at[pl.ds(off, CHK)], di_v, sem)
                ca.wait()
                cb.wait()
                pltpu.async_copy(y_hbm.at[gi_v], rows_v, sem).wait()
                pltpu.async_copy(rows_v, acc_sh.at[di_v], ss,
                                 add=True).wait()

            plsc.subcore_barrier()
            pltpu.async_copy(acc_sh.at[pl.ds(base, TROWS)],
                             out_hbm.at[pl.ds(r * RGN + base, TROWS)],
                             sem).wait()


@functools.lru_cache(maxsize=None)
def _sc_scatter():
    return functools.partial(
        pl.kernel,
        out_type=jax.ShapeDtypeStruct((NP, H), jnp.float32),
        mesh=_mesh(),
        scratch_types=[
            pltpu.VMEM((CHK,), jnp.int32),
            pltpu.VMEM((CHK,), jnp.int32),
            pltpu.VMEM((CHK, H), jnp.float32),
            pltpu.VMEM((16,), jnp.int32),
            pltpu.VMEM_SHARED((RGN_P, H), jnp.float32),
            pltpu.SemaphoreType.DMA,
            pltpu.SemaphoreType.DMA,
        ],
    )(_scatter_body)


def _setup_body(pat_hbm, spos_hbm, sval_hbm, out_hbm, pat_v, pos_v, val_v,
                sem, ss):
    c = lax.axis_index("c")
    s = lax.axis_index("s")

    @pl.when(c == 0)
    def _core0():
        pltpu.async_copy(pat_hbm, pat_v, sem).wait()

        @pl.loop(s, EP // SCH, step=NSUB)
        def _fill(b):
            pltpu.async_copy(pat_v, out_hbm.at[pl.ds(b * SCH, SCH)],
                             sem).wait()

        plsc.subcore_barrier()

        @pl.loop(s * SCH, EPS, step=NSUB * SCH)
        def _scat(off):
            off = pl.multiple_of(off, SCH)
            ca = pltpu.async_copy(spos_hbm.at[pl.ds(off, SCH)], pos_v, sem)
            cb = pltpu.async_copy(sval_hbm.at[pl.ds(off, SCH)], val_v, sem)
            ca.wait()
            cb.wait()
            pltpu.async_copy(val_v, out_hbm.at[pos_v], ss).wait()


@functools.lru_cache(maxsize=None)
def _sc_setup():
    return functools.partial(
        pl.kernel,
        out_type=jax.ShapeDtypeStruct((EP,), jnp.int32),
        mesh=_mesh(),
        scratch_types=[
            pltpu.VMEM((SCH,), jnp.int32),
            pltpu.VMEM((SCH,), jnp.int32),
            pltpu.VMEM((SCH,), jnp.int32),
            pltpu.SemaphoreType.DMA,
            pltpu.SemaphoreType.DMA,
        ],
    )(_setup_body)


@functools.lru_cache(maxsize=None)
def _make_gather(m, chk):
    nch = m // chk

    def body(tab_hbm, ids_hbm, out_hbm, ids_v, rows_v, sem):
        c = lax.axis_index("c")
        s = lax.axis_index("s")
        w = s * NCORES + c

        @pl.loop(w, nch, step=NW)
        def _chunk(j):
            off = j * chk
            pltpu.async_copy(ids_hbm.at[pl.ds(off, chk)], ids_v, sem).wait()
            pltpu.async_copy(tab_hbm.at[ids_v], rows_v, sem).wait()
            pltpu.async_copy(rows_v, out_hbm.at[pl.ds(off, chk)], sem).wait()

    return functools.partial(
        pl.kernel,
        out_type=jax.ShapeDtypeStruct((m, H), jnp.float32),
        mesh=_mesh(),
        scratch_types=[
            pltpu.VMEM((chk,), jnp.int32),
            pltpu.VMEM((chk, H), jnp.float32),
            pltpu.SemaphoreType.DMA,
        ],
    )(body)


def _gru_call(inc, h, res, Wih, Whh, bih, bhh, Wn, bn, *, final):
    has_res = res is not None
    has_y = not final
    nout = N if final else NP
    bspec = pl.BlockSpec((BLK, H), lambda i: (i, 0))

    def full(shape):
        return pl.BlockSpec(shape, lambda i: tuple(0 for _ in shape))

    in_specs = [bspec, bspec]
    args = [inc, h]
    if has_res:
        in_specs.append(bspec)
        args.append(res)
    in_specs += [full(Wih.shape), full((H, 3 * H)), full((1, 3 * H)),
                 full((1, 3 * H))]
    args += [Wih, Whh, bih.reshape(1, 3 * H), bhh.reshape(1, 3 * H)]
    if has_y:
        in_specs += [full((NTYPES, H, H)), full((NTYPES, 1, H))]
        args += [Wn, bn.reshape(NTYPES, 1, H)]
    out_shape = [jax.ShapeDtypeStruct((nout, H), jnp.float32)]
    out_specs = [bspec]
    if has_y:
        out_shape.append(jax.ShapeDtypeStruct((NTYPES, NP, H), jnp.float32))
        out_specs.append(pl.BlockSpec((NTYPES, BLK, H), lambda i: (0, i, 0)))

    def body(*refs):
        k = 2
        inc_r, h_r = refs[0], refs[1]
        res_r = None
        if has_res:
            res_r = refs[k]
            k += 1
        wih_r, whh_r, bih_r, bhh_r = refs[k:k + 4]
        k += 4
        if has_y:
            wn_r, bn_r = refs[k:k + 2]
            k += 2
        ho_r = refs[k]
        y_r = refs[k + 1] if has_y else None

        hh = h_r[...]
        wih = wih_r[...]
        gi = jnp.dot(inc_r[...], wih[:H], preferred_element_type=jnp.float32)
        if has_res:
            gi = gi + jnp.dot(res_r[...], wih[H:],
                              preferred_element_type=jnp.float32)
        gi = gi + bih_r[...]
        gh = jnp.dot(hh, whh_r[...], preferred_element_type=jnp.float32)
        gh = gh + bhh_r[...]
        rg = jax.nn.sigmoid(gi[:, :H] + gh[:, :H])
        zg = jax.nn.sigmoid(gi[:, H:2 * H] + gh[:, H:2 * H])
        ng = jnp.tanh(gi[:, 2 * H:] + rg * gh[:, 2 * H:])
        hp = (1.0 - zg) * ng + zg * hh
        ho_r[...] = hp
        if has_y:
            wn = wn_r[...]
            bnv = bn_r[...]
            for e in range(NTYPES):
                y_r[e] = jnp.dot(hp, wn[e],
                                 preferred_element_type=jnp.float32) + bnv[e]

    return pl.pallas_call(body, grid=(NBLK,), in_specs=in_specs,
                          out_specs=out_specs, out_shape=out_shape)(*args)


def _y0_call(h0, Wn, bn):
    def full(shape):
        return pl.BlockSpec(shape, lambda i: tuple(0 for _ in shape))

    def body(h_r, wn_r, bn_r, y_r):
        hh = h_r[...]
        wn = wn_r[...]
        bnv = bn_r[...]
        for e in range(NTYPES):
            y_r[e] = jnp.dot(hh, wn[e],
                             preferred_element_type=jnp.float32) + bnv[e]

    return pl.pallas_call(
        body, grid=(NBLK,),
        in_specs=[pl.BlockSpec((BLK, H), lambda i: (i, 0)),
                  full((NTYPES, H, H)), full((NTYPES, 1, H))],
        out_specs=pl.BlockSpec((NTYPES, BLK, H), lambda i: (0, i, 0)),
        out_shape=jax.ShapeDtypeStruct((NTYPES, NP, H), jnp.float32),
    )(h0, Wn, bn.reshape(NTYPES, 1, H))


def kernel(emb, msg_W, msg_b, gru_Wih_a, gru_Wih_b, gru_Whh, gru_bih,
           gru_bhh, node_type_ids, edges_ast, edges_var, var_node_ids,
           pred_node_ids):
    i32 = jnp.int32
    ids_p = jnp.concatenate([node_type_ids.astype(i32),
                             jnp.zeros((NP - N,), i32)])
    src = jnp.concatenate([edges_ast[:, 0], edges_ast[:, 1],
                           edges_var[:, 0], edges_var[:, 1]]).astype(i32)
    dst = jnp.concatenate([edges_ast[:, 1], edges_ast[:, 0],
                           edges_var[:, 1], edges_var[:, 0]]).astype(i32)
    tvec = jnp.concatenate([
        jnp.full((E_AST,), 0, i32), jnp.full((E_AST,), 1, i32),
        jnp.full((E_VAR,), 2, i32), jnp.full((E_VAR,), 3, i32)])
    gidx = tvec * NP + src

    key = dst // RGN
    onehot = (key[:, None] == jnp.arange(NRGN, dtype=i32)[None, :]).astype(i32)
    ranks = jnp.cumsum(onehot, axis=0) - 1
    rank = jnp.take_along_axis(ranks, key[:, None], axis=1)[:, 0]
    counts = ranks[-1] + 1
    pcounts = ((counts + CHKB - 1) // CHKB) * CHKB
    offs = jnp.concatenate([jnp.zeros((1,), i32),
                            jnp.cumsum(pcounts).astype(i32)])
    offs = offs.at[NRGN].set(EP)
    pos = offs[key] + rank
    spos = jnp.concatenate([pos, jnp.full((EPS - E,), EP - 1, i32)])
    sval = jnp.concatenate([gidx * PACK + (dst - key * RGN),
                            jnp.full((EPS - E,), DUMP, i32)])
    pat = jnp.full((SCH,), DUMP, i32)
    packed = _sc_setup()(pat, spos, sval)
    gidx_p = packed // PACK
    dloc_p = packed % PACK
    offs16 = jnp.zeros((16,), i32).at[:NRGN + 1].set(offs)
    zer = jnp.zeros((TROWS, H), jnp.float32)

    layers = []
    for li, t in enumerate(LAYER_TIMESTEPS):
        layers += [li] * t
    nsteps = len(layers)

    h0 = _make_gather(NP, 512)(emb, ids_p)
    yflat = _y0_call(h0, msg_W[0], msg_b[0]).reshape(NTYPES * NP, H)

    h = h0
    s1 = None
    hfin = None
    for t in range(nsteps):
        l = layers[t]
        inc = _sc_scatter()(yflat, gidx_p, dloc_p, offs16, zer)
        if l % 2 == 0:
            Wih, res = gru_Wih_a[l // 2], None
        else:
            Wih = gru_Wih_b[l // 2]
            res = h0 if l == 1 else s1
        if t + 1 < nsteps:
            ln = layers[t + 1]
            h, y = _gru_call(inc, h, res, Wih, gru_Whh[l], gru_bih[l],
                             gru_bhh[l], msg_W[ln], msg_b[ln], final=False)
            yflat = y.reshape(NTYPES * NP, H)
        else:
            hfin = _gru_call(inc, h, res, Wih, gru_Whh[l], gru_bih[l],
                             gru_bhh[l], None, None, final=True)[0]
        if t + 1 == LAYER_TIMESTEPS[0]:
            s1 = h

    var_enc = _make_gather(16000, 400)(hfin, var_node_ids.astype(i32))
    pred_enc = _make_gather(8000, 400)(hfin, pred_node_ids.astype(i32))
    return (hfin, var_enc, pred_enc)


# speedup vs baseline: 1.0209x; 1.0209x over previous
import functools

import jax
import jax.numpy as jnp
from jax import lax
from jax.experimental import pallas as pl
from jax.experimental.pallas import tpu as pltpu
from jax.experimental.pallas import tpu_sc as plsc

H = 128
N = 50000
NP = 50176
BLK = 1568
NBLK = 32
E_AST = 50000
E_VAR = 16000
E = 2 * E_AST + 2 * E_VAR
NTYPES = 4
LAYER_TIMESTEPS = [5, 2, 5, 2]

NCORES = 2
NSUB = 16
NW = NCORES * NSUB
RGN = 6272
NRGN = 8
DUMP = RGN
RGN_P = RGN + 8
CHK = 128
CHKB = 2 * CHK
EP = 134144
NCHB = EP // CHKB
PACK = 8192
TROWS = RGN // NSUB
SCH = 512
NCS = 258
EPS = NCS * SCH

@functools.lru_cache(maxsize=None)
def _mesh():
    return plsc.VectorSubcoreMesh(core_axis_name="c", subcore_axis_name="s",
                                  num_cores=NCORES, num_subcores=NSUB)


def _scatter_body(y_hbm, idx_hbm, offs_hbm, zer_hbm, out_hbm,
                  idx_v, rows_v, offs_v, acc_sh, sem, ss):
    c = lax.axis_index("c")
    s = lax.axis_index("s")
    pltpu.async_copy(offs_hbm, offs_v, sem).wait()
    offv = offs_v[...]

    for r in range(NRGN):
        @pl.when(c == (r % NCORES))
        def _phase(r=r):
            base = s * TROWS
            pltpu.async_copy(zer_hbm, acc_sh.at[pl.ds(base, TROWS)],
                             sem).wait()
            plsc.subcore_barrier()

            @pl.loop(offv[r] + s, offv[r + 1], step=NSUB)
            def _chunk(ch):
                pltpu.async_copy(idx_hbm.at[ch], idx_v, sem).wait()
                g1 = pltpu.async_copy(y_hbm.at[idx_v.at[0]],
                                      rows_v.at[pl.ds(0, CHK)], sem)
                g2 = pltpu.async_copy(y_hbm.at[idx_v.at[1]],
                                      rows_v.at[pl.ds(CHK, CHK)], sem)
                g1.wait()
                g2.wait()
                c1 = pltpu.async_copy(rows_v.at[pl.ds(0, CHK)],
                                      acc_sh.at[idx_v.at[2]], ss, add=True)
                c2 = pltpu.async_copy(rows_v.at[pl.ds(CHK, CHK)],
                                      acc_sh.at[idx_v.at[3]], ss, add=True)
                c1.wait()
                c2.wait()

            plsc.subcore_barrier()
            pltpu.async_copy(acc_sh.at[pl.ds(base, TROWS)],
                             out_hbm.at[pl.ds(r * RGN + base, TROWS)],
                             sem).wait()


@functools.lru_cache(maxsize=None)
def _sc_scatter():
    return functools.partial(
        pl.kernel,
        out_type=jax.ShapeDtypeStruct((NP, H), jnp.float32),
        mesh=_mesh(),
        scratch_types=[
            pltpu.VMEM((4, CHK), jnp.int32),
            pltpu.VMEM((CHKB, H), jnp.float32),
            pltpu.VMEM((16,), jnp.int32),
            pltpu.VMEM_SHARED((RGN_P, H), jnp.float32),
            pltpu.SemaphoreType.DMA,
            pltpu.SemaphoreType.DMA,
        ],
    )(_scatter_body)


def _setup_body(pat_hbm, spos_hbm, sval_hbm, out_hbm, pat_v, pos_v, val_v,
                sem, ss):
    c = lax.axis_index("c")
    s = lax.axis_index("s")

    @pl.when(c == 0)
    def _core0():
        pltpu.async_copy(pat_hbm, pat_v, sem).wait()

        @pl.loop(s, EP // SCH, step=NSUB)
        def _fill(b):
            pltpu.async_copy(pat_v, out_hbm.at[pl.ds(b * SCH, SCH)],
                             sem).wait()

        plsc.subcore_barrier()

        @pl.loop(s * SCH, EPS, step=NSUB * SCH)
        def _scat(off):
            off = pl.multiple_of(off, SCH)
            ca = pltpu.async_copy(spos_hbm.at[pl.ds(off, SCH)], pos_v, sem)
            cb = pltpu.async_copy(sval_hbm.at[pl.ds(off, SCH)], val_v, sem)
            ca.wait()
            cb.wait()
            pltpu.async_copy(val_v, out_hbm.at[pos_v], ss).wait()


@functools.lru_cache(maxsize=None)
def _sc_setup():
    return functools.partial(
        pl.kernel,
        out_type=jax.ShapeDtypeStruct((EP,), jnp.int32),
        mesh=_mesh(),
        scratch_types=[
            pltpu.VMEM((SCH,), jnp.int32),
            pltpu.VMEM((SCH,), jnp.int32),
            pltpu.VMEM((SCH,), jnp.int32),
            pltpu.SemaphoreType.DMA,
            pltpu.SemaphoreType.DMA,
        ],
    )(_setup_body)


@functools.lru_cache(maxsize=None)
def _make_gather(m, chk):
    nch = m // chk

    def body(tab_hbm, ids_hbm, out_hbm, ids_v, rows_v, sem):
        c = lax.axis_index("c")
        s = lax.axis_index("s")
        w = s * NCORES + c

        @pl.loop(w, nch, step=NW)
        def _chunk(j):
            off = j * chk
            pltpu.async_copy(ids_hbm.at[pl.ds(off, chk)], ids_v, sem).wait()
            pltpu.async_copy(tab_hbm.at[ids_v], rows_v, sem).wait()
            pltpu.async_copy(rows_v, out_hbm.at[pl.ds(off, chk)], sem).wait()

    return functools.partial(
        pl.kernel,
        out_type=jax.ShapeDtypeStruct((m, H), jnp.float32),
        mesh=_mesh(),
        scratch_types=[
            pltpu.VMEM((chk,), jnp.int32),
            pltpu.VMEM((chk, H), jnp.float32),
            pltpu.SemaphoreType.DMA,
        ],
    )(body)


def _gru_call(inc, h, res, Wih, Whh, bih, bhh, Wn, bn, *, final):
    has_res = res is not None
    has_y = not final
    nout = N if final else NP
    bspec = pl.BlockSpec((BLK, H), lambda i: (i, 0))

    def full(shape):
        return pl.BlockSpec(shape, lambda i: tuple(0 for _ in shape))

    in_specs = [bspec, bspec]
    args = [inc, h]
    if has_res:
        in_specs.append(bspec)
        args.append(res)
    in_specs += [full(Wih.shape), full((H, 3 * H)), full((1, 3 * H)),
                 full((1, 3 * H))]
    args += [Wih, Whh, bih.reshape(1, 3 * H), bhh.reshape(1, 3 * H)]
    if has_y:
        in_specs += [full((NTYPES, H, H)), full((NTYPES, 1, H))]
        args += [Wn, bn.reshape(NTYPES, 1, H)]
    out_shape = [jax.ShapeDtypeStruct((nout, H), jnp.float32)]
    out_specs = [bspec]
    if has_y:
        out_shape.append(jax.ShapeDtypeStruct((NTYPES, NP, H), jnp.float32))
        out_specs.append(pl.BlockSpec((NTYPES, BLK, H), lambda i: (0, i, 0)))

    def body(*refs):
        k = 2
        inc_r, h_r = refs[0], refs[1]
        res_r = None
        if has_res:
            res_r = refs[k]
            k += 1
        wih_r, whh_r, bih_r, bhh_r = refs[k:k + 4]
        k += 4
        if has_y:
            wn_r, bn_r = refs[k:k + 2]
            k += 2
        ho_r = refs[k]
        y_r = refs[k + 1] if has_y else None

        hh = h_r[...]
        wih = wih_r[...]
        gi = jnp.dot(inc_r[...], wih[:H], preferred_element_type=jnp.float32)
        if has_res:
            gi = gi + jnp.dot(res_r[...], wih[H:],
                              preferred_element_type=jnp.float32)
        gi = gi + bih_r[...]
        gh = jnp.dot(hh, whh_r[...], preferred_element_type=jnp.float32)
        gh = gh + bhh_r[...]
        rg = jax.nn.sigmoid(gi[:, :H] + gh[:, :H])
        zg = jax.nn.sigmoid(gi[:, H:2 * H] + gh[:, H:2 * H])
        ng = jnp.tanh(gi[:, 2 * H:] + rg * gh[:, 2 * H:])
        hp = (1.0 - zg) * ng + zg * hh
        ho_r[...] = hp
        if has_y:
            wn = wn_r[...]
            bnv = bn_r[...]
            for e in range(NTYPES):
                y_r[e] = jnp.dot(hp, wn[e],
                                 preferred_element_type=jnp.float32) + bnv[e]

    return pl.pallas_call(body, grid=(NBLK,), in_specs=in_specs,
                          out_specs=out_specs, out_shape=out_shape)(*args)


def _y0_call(h0, Wn, bn):
    def full(shape):
        return pl.BlockSpec(shape, lambda i: tuple(0 for _ in shape))

    def body(h_r, wn_r, bn_r, y_r):
        hh = h_r[...]
        wn = wn_r[...]
        bnv = bn_r[...]
        for e in range(NTYPES):
            y_r[e] = jnp.dot(hh, wn[e],
                             preferred_element_type=jnp.float32) + bnv[e]

    return pl.pallas_call(
        body, grid=(NBLK,),
        in_specs=[pl.BlockSpec((BLK, H), lambda i: (i, 0)),
                  full((NTYPES, H, H)), full((NTYPES, 1, H))],
        out_specs=pl.BlockSpec((NTYPES, BLK, H), lambda i: (0, i, 0)),
        out_shape=jax.ShapeDtypeStruct((NTYPES, NP, H), jnp.float32),
    )(h0, Wn, bn.reshape(NTYPES, 1, H))


def kernel(emb, msg_W, msg_b, gru_Wih_a, gru_Wih_b, gru_Whh, gru_bih,
           gru_bhh, node_type_ids, edges_ast, edges_var, var_node_ids,
           pred_node_ids):
    i32 = jnp.int32
    ids_p = jnp.concatenate([node_type_ids.astype(i32),
                             jnp.zeros((NP - N,), i32)])
    src = jnp.concatenate([edges_ast[:, 0], edges_ast[:, 1],
                           edges_var[:, 0], edges_var[:, 1]]).astype(i32)
    dst = jnp.concatenate([edges_ast[:, 1], edges_ast[:, 0],
                           edges_var[:, 1], edges_var[:, 0]]).astype(i32)
    tvec = jnp.concatenate([
        jnp.full((E_AST,), 0, i32), jnp.full((E_AST,), 1, i32),
        jnp.full((E_VAR,), 2, i32), jnp.full((E_VAR,), 3, i32)])
    gidx = tvec * NP + src

    key = dst // RGN
    onehot = (key[:, None] == jnp.arange(NRGN, dtype=i32)[None, :]).astype(i32)
    ranks = jnp.cumsum(onehot, axis=0) - 1
    rank = jnp.take_along_axis(ranks, key[:, None], axis=1)[:, 0]
    counts = ranks[-1] + 1
    pcounts = ((counts + CHKB - 1) // CHKB) * CHKB
    offs = jnp.concatenate([jnp.zeros((1,), i32),
                            jnp.cumsum(pcounts).astype(i32)])
    offs = offs.at[NRGN].set(EP)
    pos = offs[key] + rank
    spos = jnp.concatenate([pos, jnp.full((EPS - E,), EP - 1, i32)])
    sval = jnp.concatenate([gidx * PACK + (dst - key * RGN),
                            jnp.full((EPS - E,), DUMP, i32)])
    pat = jnp.full((SCH,), DUMP, i32)
    packed = _sc_setup()(pat, spos, sval)
    idx_all = jnp.concatenate(
        [(packed // PACK).reshape(NCHB, 2, CHK),
         (packed % PACK).reshape(NCHB, 2, CHK)], axis=1)
    offs16 = jnp.zeros((16,), i32).at[:NRGN + 1].set(offs // CHKB)
    zer = jnp.zeros((TROWS, H), jnp.float32)

    layers = []
    for li, t in enumerate(LAYER_TIMESTEPS):
        layers += [li] * t
    nsteps = len(layers)

    h0 = _make_gather(NP, 512)(emb, ids_p)
    yflat = _y0_call(h0, msg_W[0], msg_b[0]).reshape(NTYPES * NP, H)

    h = h0
    s1 = None
    hfin = None
    for t in range(nsteps):
        l = layers[t]
        inc = _sc_scatter()(yflat, idx_all, offs16, zer)
        if l % 2 == 0:
            Wih, res = gru_Wih_a[l // 2], None
        else:
            Wih = gru_Wih_b[l // 2]
            res = h0 if l == 1 else s1
        if t + 1 < nsteps:
            ln = layers[t + 1]
            h, y = _gru_call(inc, h, res, Wih, gru_Whh[l], gru_bih[l],
                             gru_bhh[l], msg_W[ln], msg_b[ln], final=False)
            yflat = y.reshape(NTYPES * NP, H)
        else:
            hfin = _gru_call(inc, h, res, Wih, gru_Whh[l], gru_bih[l],
                             gru_bhh[l], None, None, final=True)[0]
        if t + 1 == LAYER_TIMESTEPS[0]:
            s1 = h

    var_enc = _make_gather(16000, 400)(hfin, var_node_ids.astype(i32))
    pred_enc = _make_gather(8000, 400)(hfin, pred_node_ids.astype(i32))
    return (hfin, var_enc, pred_enc)
